# initial kernel scaffold (unmeasured)
import jax
import jax.numpy as jnp
from jax import lax
from jax.experimental import pallas as pl
from jax.experimental.pallas import tpu as pltpu

T = 2048
D = 4096
VH = 8192

V_BLK = 512
R_BLK = 128


def _cast_body(x_ref, o_ref):
    o_ref[...] = x_ref[...].astype(jnp.bfloat16)


def _cast_x(x):
    return pl.pallas_call(
        _cast_body,
        grid=(8,),
        in_specs=[pl.BlockSpec((T // 8, D), lambda i: (i, 0))],
        out_specs=pl.BlockSpec((T // 8, D), lambda i: (i, 0)),
        out_shape=jax.ShapeDtypeStruct((T, D), jnp.bfloat16),
    )(x)


def _gemm_body(x_ref, w_ref, o_ref):
    o_ref[...] = jnp.dot(
        x_ref[...], w_ref[...].astype(jnp.bfloat16),
        preferred_element_type=jnp.float32,
    ).astype(jnp.bfloat16)


def _gemm(xb, W):
    return pl.pallas_call(
        _gemm_body,
        grid=(VH // V_BLK,),
        in_specs=[
            pl.BlockSpec((T, D), lambda j: (0, 0)),
            pl.BlockSpec((D, V_BLK), lambda j: (0, j)),
        ],
        out_specs=pl.BlockSpec((T, V_BLK), lambda j: (0, j)),
        out_shape=jax.ShapeDtypeStruct((T, VH), jnp.bfloat16),
    )(xb, W)


def _exchange_body(l_ref, o_ref, send_sem, recv_sem):
    my_x = lax.axis_index("x")
    my_y = lax.axis_index("y")
    my_z = lax.axis_index("z")
    partner = (1 - my_x, my_y, my_z)

    barrier = pltpu.get_barrier_semaphore()
    pl.semaphore_signal(
        barrier, inc=1, device_id=partner,
        device_id_type=pl.DeviceIdType.MESH,
    )
    pl.semaphore_wait(barrier, 1)

    rdma = pltpu.make_async_remote_copy(
        src_ref=l_ref,
        dst_ref=o_ref,
        send_sem=send_sem,
        recv_sem=recv_sem,
        device_id=partner,
        device_id_type=pl.DeviceIdType.MESH,
    )
    rdma.start()
    rdma.wait()


def _exchange(L):
    return pl.pallas_call(
        _exchange_body,
        in_specs=[pl.BlockSpec(memory_space=pltpu.ANY)],
        out_specs=pl.BlockSpec(memory_space=pltpu.ANY),
        out_shape=jax.ShapeDtypeStruct((T, VH), jnp.bfloat16),
        scratch_shapes=[pltpu.SemaphoreType.DMA, pltpu.SemaphoreType.DMA],
        compiler_params=pltpu.CompilerParams(collective_id=0),
    )(L)


def _softmax_body(l_ref, r_ref, o_ref):
    my_x = lax.axis_index("x")
    lf = l_ref[...].astype(jnp.float32)
    rf = r_ref[...].astype(jnp.float32)
    m = jnp.maximum(
        lf.max(axis=1, keepdims=True), rf.max(axis=1, keepdims=True)
    )
    el = jnp.exp(lf - m)
    er = jnp.exp(rf - m)
    s = el.sum(axis=1, keepdims=True) + er.sum(axis=1, keepdims=True)
    el = el / s
    er = er / s

    @pl.when(my_x == 0)
    def _():
        o_ref[:, 0:VH] = el
        o_ref[:, VH:2 * VH] = er

    @pl.when(my_x != 0)
    def _():
        o_ref[:, 0:VH] = er
        o_ref[:, VH:2 * VH] = el


def _softmax(L, R):
    return pl.pallas_call(
        _softmax_body,
        grid=(T // R_BLK,),
        in_specs=[
            pl.BlockSpec((R_BLK, VH), lambda i: (i, 0)),
            pl.BlockSpec((R_BLK, VH), lambda i: (i, 0)),
        ],
        out_specs=pl.BlockSpec((R_BLK, 2 * VH), lambda i: (i, 0)),
        out_shape=jax.ShapeDtypeStruct((T, 2 * VH), jnp.float32),
    )(L, R)


def kernel(x, W):
    xb = _cast_x(x)
    L = _gemm(xb, W)
    R = _exchange(L)
    return _softmax(L, R)


# baseline (device time: 597828 ns/iter reference)
import jax
import jax.numpy as jnp
from jax import lax
from jax.experimental import pallas as pl
from jax.experimental.pallas import tpu as pltpu

T = 2048
D = 4096
VH = 8192

V_BLK = 512
R_BLK = 128


def _cast_body(x_ref, o_ref):
    o_ref[...] = x_ref[...].astype(jnp.bfloat16)


def _cast_x(x):
    return pl.pallas_call(
        _cast_body,
        grid=(8,),
        in_specs=[pl.BlockSpec((T // 8, D), lambda i: (i, 0))],
        out_specs=pl.BlockSpec((T // 8, D), lambda i: (i, 0)),
        out_shape=jax.ShapeDtypeStruct((T, D), jnp.bfloat16),
    )(x)


def _gemm_body(x_ref, w_ref, o_ref):
    o_ref[...] = jnp.dot(
        x_ref[...], w_ref[...].astype(jnp.bfloat16),
        preferred_element_type=jnp.float32,
    ).astype(jnp.bfloat16)


def _gemm(xb, W):
    return pl.pallas_call(
        _gemm_body,
        grid=(VH // V_BLK,),
        in_specs=[
            pl.BlockSpec((T, D), lambda j: (0, 0)),
            pl.BlockSpec((D, V_BLK), lambda j: (0, j)),
        ],
        out_specs=pl.BlockSpec((T, V_BLK), lambda j: (0, j)),
        out_shape=jax.ShapeDtypeStruct((T, VH), jnp.bfloat16),
    )(xb, W)


def _exchange_body(l_ref, o_ref, send_sem, recv_sem):
    my_x = lax.axis_index("x")
    my_y = lax.axis_index("y")
    my_z = lax.axis_index("z")
    partner = (1 - my_x, my_y, my_z)

    barrier = pltpu.get_barrier_semaphore()
    pl.semaphore_signal(
        barrier, inc=1, device_id=partner,
        device_id_type=pl.DeviceIdType.MESH,
    )
    pl.semaphore_wait(barrier, 1)

    rdma = pltpu.make_async_remote_copy(
        src_ref=l_ref,
        dst_ref=o_ref,
        send_sem=send_sem,
        recv_sem=recv_sem,
        device_id=partner,
        device_id_type=pl.DeviceIdType.MESH,
    )
    rdma.start()
    rdma.wait()


def _exchange(L):
    return pl.pallas_call(
        _exchange_body,
        in_specs=[pl.BlockSpec(memory_space=pl.ANY)],
        out_specs=pl.BlockSpec(memory_space=pl.ANY),
        out_shape=jax.ShapeDtypeStruct((T, VH), jnp.bfloat16),
        scratch_shapes=[pltpu.SemaphoreType.DMA, pltpu.SemaphoreType.DMA],
        compiler_params=pltpu.CompilerParams(collective_id=0),
    )(L)


def _softmax_body(l_ref, r_ref, o_ref):
    my_x = lax.axis_index("x")
    lf = l_ref[...].astype(jnp.float32)
    rf = r_ref[...].astype(jnp.float32)
    m = jnp.maximum(
        lf.max(axis=1, keepdims=True), rf.max(axis=1, keepdims=True)
    )
    el = jnp.exp(lf - m)
    er = jnp.exp(rf - m)
    s = el.sum(axis=1, keepdims=True) + er.sum(axis=1, keepdims=True)
    el = el / s
    er = er / s

    @pl.when(my_x == 0)
    def _():
        o_ref[:, 0:VH] = el
        o_ref[:, VH:2 * VH] = er

    @pl.when(my_x != 0)
    def _():
        o_ref[:, 0:VH] = er
        o_ref[:, VH:2 * VH] = el


def _softmax(L, R):
    return pl.pallas_call(
        _softmax_body,
        grid=(T // R_BLK,),
        in_specs=[
            pl.BlockSpec((R_BLK, VH), lambda i: (i, 0)),
            pl.BlockSpec((R_BLK, VH), lambda i: (i, 0)),
        ],
        out_specs=pl.BlockSpec((R_BLK, 2 * VH), lambda i: (i, 0)),
        out_shape=jax.ShapeDtypeStruct((T, 2 * VH), jnp.float32),
    )(L, R)


def kernel(x, W):
    xb = _cast_x(x)
    L = _gemm(xb, W)
    R = _exchange(L)
    return _softmax(L, R)


# device time: 433649 ns/iter; 1.3786x vs baseline; 1.3786x over previous
import jax
import jax.numpy as jnp
from jax import lax
from jax.experimental import pallas as pl
from jax.experimental.pallas import tpu as pltpu

T = 2048
D = 4096
VH = 8192

V_BLK = 512
R_BLK = 128


def _cast_body(x_ref, o_ref):
    o_ref[...] = x_ref[...].astype(jnp.bfloat16)


def _cast_x(x):
    return pl.pallas_call(
        _cast_body,
        grid=(8,),
        in_specs=[pl.BlockSpec((T // 8, D), lambda i: (i, 0))],
        out_specs=pl.BlockSpec((T // 8, D), lambda i: (i, 0)),
        out_shape=jax.ShapeDtypeStruct((T, D), jnp.bfloat16),
    )(x)


def _gemm_body(x_ref, w_ref, o_ref):
    o_ref[...] = jnp.dot(
        x_ref[...], w_ref[...].astype(jnp.bfloat16),
        preferred_element_type=jnp.float32,
    ).astype(jnp.bfloat16)


def _gemm(xb, W):
    return pl.pallas_call(
        _gemm_body,
        grid=(VH // V_BLK,),
        in_specs=[
            pl.BlockSpec((T, D), lambda j: (0, 0)),
            pl.BlockSpec((D, V_BLK), lambda j: (0, j)),
        ],
        out_specs=pl.BlockSpec((T, V_BLK), lambda j: (0, j)),
        out_shape=jax.ShapeDtypeStruct((T, VH), jnp.bfloat16),
    )(xb, W)


C = 8
QR = T // 4
CR = QR // C


def _exchange_body(l_ref, o_ref, sx, rx, sy, ry, sz, rz):
    my_x = lax.axis_index("x")
    my_y = lax.axis_index("y")
    my_z = lax.axis_index("z")
    px = (1 - my_x, my_y, my_z)
    py = (my_x, 1 - my_y, my_z)
    pz = (my_x, my_y, 1 - my_z)
    row_me = (2 * my_y + my_z) * QR
    row_y = (2 * (1 - my_y) + my_z) * QR

    barrier = pltpu.get_barrier_semaphore()
    for p in (px, py, pz):
        pl.semaphore_signal(
            barrier, inc=1, device_id=p,
            device_id_type=pl.DeviceIdType.MESH,
        )
    pl.semaphore_wait(barrier, 3)

    def copy(src_row, dst_row, s_sem, r_sem, peer):
        return pltpu.make_async_remote_copy(
            src_ref=l_ref.at[pl.ds(src_row, CR), :] if peer is px
            else o_ref.at[pl.ds(src_row, CR), :],
            dst_ref=o_ref.at[pl.ds(dst_row, CR), :],
            send_sem=s_sem,
            recv_sem=r_sem,
            device_id=peer,
            device_id_type=pl.DeviceIdType.MESH,
        )

    xr = [copy(row_me + k * CR, row_me + k * CR, sx.at[k], rx.at[k], px)
          for k in range(C)]
    for r in xr:
        r.start()

    yr, zr = [], []
    for k in range(C):
        xr[k].wait_recv()
        r1 = copy(row_me + k * CR, row_me + k * CR, sy.at[k], ry.at[k], py)
        r1.start()
        yr.append(r1)
        r2 = copy(row_me + k * CR, row_me + k * CR,
                  sz.at[k], rz.at[k], pz)
        r2.start()
        zr.append(r2)

    for k in range(C):
        yr[k].wait_recv()
        r2 = copy(row_y + k * CR, row_y + k * CR,
                  sz.at[C + k], rz.at[C + k], pz)
        r2.start()
        zr.append(r2)

    for r in zr:
        r.wait_recv()
    for r in xr + yr + zr:
        r.wait_send()


def _exchange(L):
    return pl.pallas_call(
        _exchange_body,
        in_specs=[pl.BlockSpec(memory_space=pl.ANY)],
        out_specs=pl.BlockSpec(memory_space=pl.ANY),
        out_shape=jax.ShapeDtypeStruct((T, VH), jnp.bfloat16),
        scratch_shapes=[
            pltpu.SemaphoreType.DMA((C,)),
            pltpu.SemaphoreType.DMA((C,)),
            pltpu.SemaphoreType.DMA((C,)),
            pltpu.SemaphoreType.DMA((C,)),
            pltpu.SemaphoreType.DMA((2 * C,)),
            pltpu.SemaphoreType.DMA((2 * C,)),
        ],
        compiler_params=pltpu.CompilerParams(collective_id=0),
    )(L)


def _softmax_body(l_ref, r_ref, o_ref):
    my_x = lax.axis_index("x")
    lf = l_ref[...].astype(jnp.float32)
    rf = r_ref[...].astype(jnp.float32)
    m = jnp.maximum(
        lf.max(axis=1, keepdims=True), rf.max(axis=1, keepdims=True)
    )
    el = jnp.exp(lf - m)
    er = jnp.exp(rf - m)
    s = el.sum(axis=1, keepdims=True) + er.sum(axis=1, keepdims=True)
    el = el / s
    er = er / s

    @pl.when(my_x == 0)
    def _():
        o_ref[:, 0:VH] = el
        o_ref[:, VH:2 * VH] = er

    @pl.when(my_x != 0)
    def _():
        o_ref[:, 0:VH] = er
        o_ref[:, VH:2 * VH] = el


def _softmax(L, R):
    return pl.pallas_call(
        _softmax_body,
        grid=(T // R_BLK,),
        in_specs=[
            pl.BlockSpec((R_BLK, VH), lambda i: (i, 0)),
            pl.BlockSpec((R_BLK, VH), lambda i: (i, 0)),
        ],
        out_specs=pl.BlockSpec((R_BLK, 2 * VH), lambda i: (i, 0)),
        out_shape=jax.ShapeDtypeStruct((T, 2 * VH), jnp.float32),
    )(L, R)


def kernel(x, W):
    xb = _cast_x(x)
    L = _gemm(xb, W)
    R = _exchange(L)
    return _softmax(L, R)


# device time: 321154 ns/iter; 1.8615x vs baseline; 1.3503x over previous
import jax
import jax.numpy as jnp
from jax import lax
from jax.experimental import pallas as pl
from jax.experimental.pallas import tpu as pltpu

T = 2048
D = 4096
VH = 8192

NB = 16
VB = VH // NB
NQ = 4
QB = NB // NQ

R_BLK = 128


def _cast_body(x_ref, o_ref):
    o_ref[...] = x_ref[...].astype(jnp.bfloat16)


def _cast_x(x):
    return pl.pallas_call(
        _cast_body,
        grid=(8,),
        in_specs=[pl.BlockSpec((T // 8, D), lambda i: (i, 0))],
        out_specs=pl.BlockSpec((T // 8, D), lambda i: (i, 0)),
        out_shape=jax.ShapeDtypeStruct((T, D), jnp.bfloat16),
    )(x)


def _fused_body(xb_ref, w_ref, l_ref, r_ref,
                wbuf, wsem, lbuf, lsem, sbuf, ssem,
                sx, rx, sy, ry, sz, rz):
    s = pl.program_id(0)
    my_x = lax.axis_index("x")
    my_y = lax.axis_index("y")
    my_z = lax.axis_index("z")
    px = (1 - my_x, my_y, my_z)
    py = (my_x, 1 - my_y, my_z)
    pz = (my_x, my_y, 1 - my_z)
    q_me = 2 * my_y + my_z
    q_y = 2 * (1 - my_y) + my_z
    j = (QB * q_me + s) % NB
    cur = s % 2

    def w_dma(step, slot):
        jn = (QB * q_me + step) % NB
        return pltpu.make_async_copy(
            w_ref.at[:, pl.ds(jn * VB, VB)], wbuf.at[slot], wsem.at[slot]
        )

    def fwd(blk_idx, src_ref, s_sem, r_sem, peer):
        return pltpu.make_async_remote_copy(
            src_ref=src_ref,
            dst_ref=r_ref.at[blk_idx],
            send_sem=s_sem,
            recv_sem=r_sem,
            device_id=peer,
            device_id_type=pl.DeviceIdType.MESH,
        )

    @pl.when(s == 0)
    def _():
        barrier = pltpu.get_barrier_semaphore()
        for p in (px, py, pz):
            pl.semaphore_signal(
                barrier, inc=1, device_id=p,
                device_id_type=pl.DeviceIdType.MESH,
            )
        pl.semaphore_wait(barrier, 3)
        w_dma(0, 0).start()

    @pl.when(s + 1 < NB)
    def _():
        w_dma(s + 1, (s + 1) % 2).start()

    w_dma(s, cur).wait()
    blk = jnp.dot(
        xb_ref[...], wbuf[cur].astype(jnp.bfloat16),
        preferred_element_type=jnp.float32,
    ).astype(jnp.bfloat16)

    @pl.when(s < NQ)
    def _():
        sbuf[pl.ds(s, 1)] = blk[None]
        pltpu.make_async_copy(sbuf.at[s], l_ref.at[j], ssem.at[s]).start()
        fwd(j, sbuf.at[s], sx.at[s], rx.at[s], px).start()

    @pl.when(s >= NQ)
    def _():
        @pl.when(s >= NQ + 2)
        def _():
            pltpu.make_async_copy(
                lbuf.at[cur], l_ref.at[j], lsem.at[cur]
            ).wait()

        lbuf[pl.ds(cur, 1)] = blk[None]
        pltpu.make_async_copy(
            lbuf.at[cur], l_ref.at[j], lsem.at[cur]
        ).start()

    @pl.when((s >= NQ) & (s < 2 * NQ))
    def _():
        k = s - NQ
        jk = QB * q_me + k
        fwd(jk, sbuf.at[k], sx.at[k], rx.at[k], px).wait_recv()
        fwd(jk, r_ref.at[jk], sy.at[k], ry.at[k], py).start()
        fwd(jk, r_ref.at[jk], sz.at[k], rz.at[k], pz).start()

    @pl.when((s >= 9) & (s < 9 + NQ))
    def _():
        k = s - 9
        jyk = QB * q_y + k
        fwd(jyk, r_ref.at[jyk], sy.at[k], ry.at[k], py).wait_recv()
        fwd(jyk, r_ref.at[jyk], sz.at[NQ + k], rz.at[NQ + k], pz).start()

    @pl.when(s == NB - 1)
    def _():
        for k in range(2 * NQ):
            qq = q_me if k < NQ else q_y
            jz = QB * qq + (k % NQ)
            fwd(jz, r_ref.at[jz], sz.at[k], rz.at[k], pz).wait_recv()
        for k in range(NQ):
            jk = QB * q_me + k
            fwd(jk, sbuf.at[k], sx.at[k], rx.at[k], px).wait_send()
            fwd(jk, r_ref.at[jk], sy.at[k], ry.at[k], py).wait_send()
            fwd(jk, r_ref.at[jk], sz.at[k], rz.at[k], pz).wait_send()
            jyk = QB * q_y + k
            fwd(jyk, r_ref.at[jyk], sz.at[NQ + k], rz.at[NQ + k],
                pz).wait_send()
            pltpu.make_async_copy(
                sbuf.at[k], l_ref.at[k], ssem.at[k]
            ).wait()
        for i in range(2):
            pltpu.make_async_copy(
                lbuf.at[i], l_ref.at[i], lsem.at[i]
            ).wait()


def _fused_gemm_exchange(xb, W):
    blk3 = jax.ShapeDtypeStruct((NB, T, VB), jnp.bfloat16)
    return pl.pallas_call(
        _fused_body,
        grid=(NB,),
        in_specs=[
            pl.BlockSpec(memory_space=pltpu.MemorySpace.VMEM),
            pl.BlockSpec(memory_space=pl.ANY),
        ],
        out_specs=[
            pl.BlockSpec(memory_space=pl.ANY),
            pl.BlockSpec(memory_space=pl.ANY),
        ],
        out_shape=[blk3, blk3],
        scratch_shapes=[
            pltpu.VMEM((2, D, VB), jnp.float32),
            pltpu.SemaphoreType.DMA((2,)),
            pltpu.VMEM((2, T, VB), jnp.bfloat16),
            pltpu.SemaphoreType.DMA((2,)),
            pltpu.VMEM((NQ, T, VB), jnp.bfloat16),
            pltpu.SemaphoreType.DMA((NQ,)),
            pltpu.SemaphoreType.DMA((NQ,)),
            pltpu.SemaphoreType.DMA((NQ,)),
            pltpu.SemaphoreType.DMA((NQ,)),
            pltpu.SemaphoreType.DMA((NQ,)),
            pltpu.SemaphoreType.DMA((2 * NQ,)),
            pltpu.SemaphoreType.DMA((2 * NQ,)),
        ],
        compiler_params=pltpu.CompilerParams(
            collective_id=0, dimension_semantics=("arbitrary",)
        ),
    )(xb, W)


def _softmax_body(l_ref, r_ref, o_ref):
    my_x = lax.axis_index("x")
    lf = l_ref[...].astype(jnp.float32)
    rf = r_ref[...].astype(jnp.float32)
    m = jnp.maximum(
        lf.max(axis=(0, 2), keepdims=True), rf.max(axis=(0, 2), keepdims=True)
    )
    el = jnp.exp(lf - m)
    er = jnp.exp(rf - m)
    s = el.sum(axis=(0, 2), keepdims=True) + er.sum(axis=(0, 2), keepdims=True)
    el = el / s
    er = er / s

    @pl.when(my_x == 0)
    def _():
        for j in range(NB):
            o_ref[:, j * VB:(j + 1) * VB] = el[j]
            o_ref[:, VH + j * VB:VH + (j + 1) * VB] = er[j]

    @pl.when(my_x != 0)
    def _():
        for j in range(NB):
            o_ref[:, j * VB:(j + 1) * VB] = er[j]
            o_ref[:, VH + j * VB:VH + (j + 1) * VB] = el[j]


def _softmax(L, R):
    return pl.pallas_call(
        _softmax_body,
        grid=(T // R_BLK,),
        in_specs=[
            pl.BlockSpec((NB, R_BLK, VB), lambda i: (0, i, 0)),
            pl.BlockSpec((NB, R_BLK, VB), lambda i: (0, i, 0)),
        ],
        out_specs=pl.BlockSpec((R_BLK, 2 * VH), lambda i: (i, 0)),
        out_shape=jax.ShapeDtypeStruct((T, 2 * VH), jnp.float32),
    )(L, R)


def kernel(x, W):
    xb = _cast_x(x)
    L, R = _fused_gemm_exchange(xb, W)
    return _softmax(L, R)


# device time: 287219 ns/iter; 2.0814x vs baseline; 1.1182x over previous
import jax
import jax.numpy as jnp
from jax import lax
from jax.experimental import pallas as pl
from jax.experimental.pallas import tpu as pltpu

T = 2048
D = 4096
VH = 8192

NB = 16
VB = VH // NB
NQ = 4
QB = NB // NQ

R_BLK = 128


def _cast_body(x_ref, o_ref):
    o_ref[...] = x_ref[...].astype(jnp.bfloat16)


def _cast_x(x):
    return pl.pallas_call(
        _cast_body,
        grid=(8,),
        in_specs=[pl.BlockSpec((T // 8, D), lambda i: (i, 0))],
        out_specs=pl.BlockSpec((T // 8, D), lambda i: (i, 0)),
        out_shape=jax.ShapeDtypeStruct((T, D), jnp.bfloat16),
    )(x)


def _fused_body(xb_ref, w_ref, l_ref, r_ref,
                wbuf, wsem, lbuf, lsem, sbuf, ssem,
                sx, rx, sy, ry, sz, rz):
    s = pl.program_id(0)
    my_x = lax.axis_index("x")
    my_y = lax.axis_index("y")
    my_z = lax.axis_index("z")
    px = (1 - my_x, my_y, my_z)
    py = (my_x, 1 - my_y, my_z)
    pz = (my_x, my_y, 1 - my_z)
    q_me = 2 * my_y + my_z
    q_y = 2 * (1 - my_y) + my_z
    q_z = 2 * my_y + (1 - my_z)
    j = (QB * q_me + s) % NB
    cur = s % 2

    def w_dma(step, slot):
        jn = (QB * q_me + step) % NB
        return pltpu.make_async_copy(
            w_ref.at[:, pl.ds(jn * VB, VB)], wbuf.at[slot], wsem.at[slot]
        )

    def fwd(blk_idx, src_ref, s_sem, r_sem, peer):
        return pltpu.make_async_remote_copy(
            src_ref=src_ref,
            dst_ref=r_ref.at[blk_idx],
            send_sem=s_sem,
            recv_sem=r_sem,
            device_id=peer,
            device_id_type=pl.DeviceIdType.MESH,
        )

    @pl.when(s == 0)
    def _():
        barrier = pltpu.get_barrier_semaphore()
        for p in (px, py, pz):
            pl.semaphore_signal(
                barrier, inc=1, device_id=p,
                device_id_type=pl.DeviceIdType.MESH,
            )
        pl.semaphore_wait(barrier, 3)
        w_dma(0, 0).start()

    @pl.when(s + 1 < NB)
    def _():
        w_dma(s + 1, (s + 1) % 2).start()

    w_dma(s, cur).wait()
    blk = jnp.dot(
        xb_ref[...], wbuf[cur].astype(jnp.bfloat16),
        preferred_element_type=jnp.float32,
    ).astype(jnp.bfloat16)

    @pl.when(s < NQ)
    def _():
        sbuf[pl.ds(s, 1)] = blk[None]
        pltpu.make_async_copy(sbuf.at[s], l_ref.at[j], ssem.at[s]).start()
        fwd(j, sbuf.at[s], sx.at[s], rx.at[s], px).start()

    @pl.when(s >= NQ)
    def _():
        @pl.when(s >= NQ + 2)
        def _():
            pltpu.make_async_copy(
                lbuf.at[cur], l_ref.at[j], lsem.at[cur]
            ).wait()

        lbuf[pl.ds(cur, 1)] = blk[None]
        pltpu.make_async_copy(
            lbuf.at[cur], l_ref.at[j], lsem.at[cur]
        ).start()

    @pl.when((s >= NQ) & (s < 2 * NQ))
    def _():
        k = s - NQ
        jk = QB * q_me + k
        fwd(jk, sbuf.at[k], sx.at[k], rx.at[k], px).wait_recv()
        fwd(jk, r_ref.at[jk], sy.at[k], ry.at[k], py).start()
        fwd(jk, r_ref.at[jk], sz.at[k], rz.at[k], pz).start()

    @pl.when((s >= 9) & (s < 11))
    def _():
        k = s - 9
        jyk = QB * q_y + k
        fwd(jyk, r_ref.at[jyk], sy.at[k], ry.at[k], py).wait_recv()
        fwd(jyk, r_ref.at[jyk], sz.at[NQ + k], rz.at[NQ + k], pz).start()

    @pl.when((s >= 12) & (s < 14))
    def _():
        k = s - 12
        jzk = QB * q_z + 2 + k
        fwd(jzk, r_ref.at[jzk], sz.at[2 + k], rz.at[2 + k], pz).wait_recv()
        fwd(jzk, r_ref.at[jzk], sy.at[NQ + k], ry.at[NQ + k], py).start()

    @pl.when(s == NB - 1)
    def _():
        for k in range(2):
            jz = QB * q_z + k
            fwd(jz, r_ref.at[jz], sz.at[k], rz.at[k], pz).wait_recv()
            jy = QB * q_y + 2 + k
            fwd(jy, r_ref.at[jy], sy.at[2 + k], ry.at[2 + k],
                py).wait_recv()
            jdz = QB * (2 * (1 - my_y) + (1 - my_z)) + k
            fwd(jdz, r_ref.at[jdz], sz.at[NQ + k], rz.at[NQ + k],
                pz).wait_recv()
            jdy = QB * (2 * (1 - my_y) + (1 - my_z)) + 2 + k
            fwd(jdy, r_ref.at[jdy], sy.at[NQ + k], ry.at[NQ + k],
                py).wait_recv()
        for k in range(NQ):
            jk = QB * q_me + k
            fwd(jk, sbuf.at[k], sx.at[k], rx.at[k], px).wait_send()
            fwd(jk, r_ref.at[jk], sy.at[k], ry.at[k], py).wait_send()
            fwd(jk, r_ref.at[jk], sz.at[k], rz.at[k], pz).wait_send()
            pltpu.make_async_copy(
                sbuf.at[k], l_ref.at[k], ssem.at[k]
            ).wait()
        for k in range(2):
            jyk = QB * q_y + k
            fwd(jyk, r_ref.at[jyk], sz.at[NQ + k], rz.at[NQ + k],
                pz).wait_send()
            jzk = QB * q_z + 2 + k
            fwd(jzk, r_ref.at[jzk], sy.at[NQ + k], ry.at[NQ + k],
                py).wait_send()
        for i in range(2):
            pltpu.make_async_copy(
                lbuf.at[i], l_ref.at[i], lsem.at[i]
            ).wait()


def _fused_gemm_exchange(xb, W):
    blk3 = jax.ShapeDtypeStruct((NB, T, VB), jnp.bfloat16)
    return pl.pallas_call(
        _fused_body,
        grid=(NB,),
        in_specs=[
            pl.BlockSpec(memory_space=pltpu.MemorySpace.VMEM),
            pl.BlockSpec(memory_space=pl.ANY),
        ],
        out_specs=[
            pl.BlockSpec(memory_space=pl.ANY),
            pl.BlockSpec(memory_space=pl.ANY),
        ],
        out_shape=[blk3, blk3],
        scratch_shapes=[
            pltpu.VMEM((2, D, VB), jnp.float32),
            pltpu.SemaphoreType.DMA((2,)),
            pltpu.VMEM((2, T, VB), jnp.bfloat16),
            pltpu.SemaphoreType.DMA((2,)),
            pltpu.VMEM((NQ, T, VB), jnp.bfloat16),
            pltpu.SemaphoreType.DMA((NQ,)),
            pltpu.SemaphoreType.DMA((NQ,)),
            pltpu.SemaphoreType.DMA((NQ,)),
            pltpu.SemaphoreType.DMA((NQ + 2,)),
            pltpu.SemaphoreType.DMA((NQ + 2,)),
            pltpu.SemaphoreType.DMA((NQ + 2,)),
            pltpu.SemaphoreType.DMA((NQ + 2,)),
        ],
        compiler_params=pltpu.CompilerParams(
            collective_id=0, dimension_semantics=("arbitrary",)
        ),
    )(xb, W)


def _softmax_body(l_ref, r_ref, o_ref):
    my_x = lax.axis_index("x")
    lf = l_ref[...].astype(jnp.float32)
    rf = r_ref[...].astype(jnp.float32)
    m = jnp.maximum(
        lf.max(axis=(0, 2), keepdims=True), rf.max(axis=(0, 2), keepdims=True)
    )
    el = jnp.exp(lf - m)
    er = jnp.exp(rf - m)
    s = el.sum(axis=(0, 2), keepdims=True) + er.sum(axis=(0, 2), keepdims=True)
    el = el / s
    er = er / s

    @pl.when(my_x == 0)
    def _():
        for j in range(NB):
            o_ref[:, j * VB:(j + 1) * VB] = el[j]
            o_ref[:, VH + j * VB:VH + (j + 1) * VB] = er[j]

    @pl.when(my_x != 0)
    def _():
        for j in range(NB):
            o_ref[:, j * VB:(j + 1) * VB] = er[j]
            o_ref[:, VH + j * VB:VH + (j + 1) * VB] = el[j]


def _softmax(L, R):
    return pl.pallas_call(
        _softmax_body,
        grid=(T // R_BLK,),
        in_specs=[
            pl.BlockSpec((NB, R_BLK, VB), lambda i: (0, i, 0)),
            pl.BlockSpec((NB, R_BLK, VB), lambda i: (0, i, 0)),
        ],
        out_specs=pl.BlockSpec((R_BLK, 2 * VH), lambda i: (i, 0)),
        out_shape=jax.ShapeDtypeStruct((T, 2 * VH), jnp.float32),
    )(L, R)


def kernel(x, W):
    xb = _cast_x(x)
    L, R = _fused_gemm_exchange(xb, W)
    return _softmax(L, R)


# device time: 268416 ns/iter; 2.2272x vs baseline; 1.0701x over previous
import jax
import jax.numpy as jnp
from jax import lax
from jax.experimental import pallas as pl
from jax.experimental.pallas import tpu as pltpu

T = 2048
D = 4096
VH = 8192

NB = 16
VB = VH // NB
NQ = 4
QB = NB // NQ
GR = T // 2
NG = 2 * QB

R_BLK = 128


def _cast_body(x_ref, o_ref):
    o_ref[...] = x_ref[...].astype(jnp.bfloat16)


def _cast_x(x):
    return pl.pallas_call(
        _cast_body,
        grid=(8,),
        in_specs=[pl.BlockSpec((T // 8, D), lambda i: (i, 0))],
        out_specs=pl.BlockSpec((T // 8, D), lambda i: (i, 0)),
        out_shape=jax.ShapeDtypeStruct((T, D), jnp.bfloat16),
    )(x)


def _fused_body(xb_ref, w_ref, l_ref, r_ref,
                wbuf, wsem, lbuf, lsem, sbuf, ssem,
                sx, rx, sy, ry, sz, rz):
    s = pl.program_id(0)
    my_x = lax.axis_index("x")
    my_y = lax.axis_index("y")
    my_z = lax.axis_index("z")
    px = (1 - my_x, my_y, my_z)
    py = (my_x, 1 - my_y, my_z)
    pz = (my_x, my_y, 1 - my_z)
    q_me = 2 * my_y + my_z
    q_y = 2 * (1 - my_y) + my_z
    q_z = 2 * my_y + (1 - my_z)
    j = (QB * q_me + s) % NB
    cur = s % 2

    def w_dma(step, slot):
        jn = (QB * q_me + step) % NB
        return pltpu.make_async_copy(
            w_ref.at[:, pl.ds(jn * VB, VB)], wbuf.at[slot], wsem.at[slot]
        )

    def gref(q, g):
        return r_ref.at[QB * q + g // 2, pl.ds((g % 2) * GR, GR), :]

    def fwd(dst_ref, src_ref, s_sem, r_sem, peer):
        return pltpu.make_async_remote_copy(
            src_ref=src_ref,
            dst_ref=dst_ref,
            send_sem=s_sem,
            recv_sem=r_sem,
            device_id=peer,
            device_id_type=pl.DeviceIdType.MESH,
        )

    @pl.when(s == 0)
    def _():
        barrier = pltpu.get_barrier_semaphore()
        for p in (px, py, pz):
            pl.semaphore_signal(
                barrier, inc=1, device_id=p,
                device_id_type=pl.DeviceIdType.MESH,
            )
        pl.semaphore_wait(barrier, 3)
        w_dma(0, 0).start()

    @pl.when(s + 1 < NB)
    def _():
        w_dma(s + 1, (s + 1) % 2).start()


    @pl.when((s >= 2) & (s < 2 + NG))
    def _():
        k = s - 2
        g = gref(q_me, k)
        fwd(g, g, sx.at[k], rx.at[k], px).wait_recv()
        fwd(g, g, sy.at[k], ry.at[k], py).start()
        fwd(g, g, sz.at[k], rz.at[k], pz).start()

    @pl.when((s >= 10) & (s < 12))
    def _():
        for h in range(2):
            g = 2 * (s - 10) + h
            gr = gref(q_y, g)
            fwd(gr, gr, sy.at[g], ry.at[g], py).wait_recv()
            fwd(gr, gr, sz.at[NG + g], rz.at[NG + g], pz).start()

    @pl.when((s >= 12) & (s < 14))
    def _():
        for h in range(2):
            g = 4 + 2 * (s - 12) + h
            gr = gref(q_z, g)
            fwd(gr, gr, sz.at[g], rz.at[g], pz).wait_recv()
            fwd(gr, gr, sy.at[NG + g - 4], ry.at[NG + g - 4], py).start()


    w_dma(s, cur).wait()
    blk = jnp.dot(
        xb_ref[...], wbuf[cur].astype(jnp.bfloat16),
        preferred_element_type=jnp.float32,
    ).astype(jnp.bfloat16)

    @pl.when(s < NQ)
    def _():
        sbuf[pl.ds(s, 1)] = blk[None]
        pltpu.make_async_copy(sbuf.at[s], l_ref.at[j], ssem.at[s]).start()
        for h in range(2):
            k = 2 * s + h
            fwd(gref(q_me, k), sbuf.at[s, pl.ds(h * GR, GR), :],
                sx.at[k], rx.at[k], px).start()

    @pl.when(s >= NQ)
    def _():
        @pl.when(s >= NQ + 2)
        def _():
            pltpu.make_async_copy(
                lbuf.at[cur], l_ref.at[j], lsem.at[cur]
            ).wait()

        lbuf[pl.ds(cur, 1)] = blk[None]
        pltpu.make_async_copy(
            lbuf.at[cur], l_ref.at[j], lsem.at[cur]
        ).start()

    @pl.when(s == NB - 1)
    def _():
        q_d = 2 * (1 - my_y) + (1 - my_z)
        for g in range(4):
            gr = gref(q_z, g)
            fwd(gr, gr, sz.at[g], rz.at[g], pz).wait_recv()
            gr = gref(q_y, 4 + g)
            fwd(gr, gr, sy.at[4 + g], ry.at[4 + g], py).wait_recv()
            gr = gref(q_d, g)
            fwd(gr, gr, sz.at[NG + g], rz.at[NG + g], pz).wait_recv()
            gr = gref(q_d, 4 + g)
            fwd(gr, gr, sy.at[NG + g], ry.at[NG + g], py).wait_recv()
        for k in range(NG):
            gr = gref(q_me, k)
            fwd(gr, gr, sx.at[k], rx.at[k], px).wait_send()
            fwd(gr, gr, sy.at[k], ry.at[k], py).wait_send()
            fwd(gr, gr, sz.at[k], rz.at[k], pz).wait_send()
        for g in range(4):
            gr = gref(q_y, g)
            fwd(gr, gr, sz.at[NG + g], rz.at[NG + g], pz).wait_send()
            gr = gref(q_z, 4 + g)
            fwd(gr, gr, sy.at[NG + g], ry.at[NG + g], py).wait_send()
        for k in range(NQ):
            pltpu.make_async_copy(
                sbuf.at[k], l_ref.at[k], ssem.at[k]
            ).wait()
        for i in range(2):
            pltpu.make_async_copy(
                lbuf.at[i], l_ref.at[i], lsem.at[i]
            ).wait()


def _fused_gemm_exchange(xb, W):
    blk3 = jax.ShapeDtypeStruct((NB, T, VB), jnp.bfloat16)
    return pl.pallas_call(
        _fused_body,
        grid=(NB,),
        in_specs=[
            pl.BlockSpec(memory_space=pltpu.MemorySpace.VMEM),
            pl.BlockSpec(memory_space=pl.ANY),
        ],
        out_specs=[
            pl.BlockSpec(memory_space=pl.ANY),
            pl.BlockSpec(memory_space=pl.ANY),
        ],
        out_shape=[blk3, blk3],
        scratch_shapes=[
            pltpu.VMEM((2, D, VB), jnp.float32),
            pltpu.SemaphoreType.DMA((2,)),
            pltpu.VMEM((2, T, VB), jnp.bfloat16),
            pltpu.SemaphoreType.DMA((2,)),
            pltpu.VMEM((NQ, T, VB), jnp.bfloat16),
            pltpu.SemaphoreType.DMA((NQ,)),
            pltpu.SemaphoreType.DMA((NG,)),
            pltpu.SemaphoreType.DMA((NG,)),
            pltpu.SemaphoreType.DMA((NG + 4,)),
            pltpu.SemaphoreType.DMA((NG + 4,)),
            pltpu.SemaphoreType.DMA((NG + 4,)),
            pltpu.SemaphoreType.DMA((NG + 4,)),
        ],
        compiler_params=pltpu.CompilerParams(
            collective_id=0, dimension_semantics=("arbitrary",)
        ),
    )(xb, W)


def _softmax_body(l_ref, r_ref, o_ref):
    my_x = lax.axis_index("x")
    lf = l_ref[...].astype(jnp.float32)
    rf = r_ref[...].astype(jnp.float32)
    m = jnp.maximum(
        lf.max(axis=(0, 2), keepdims=True), rf.max(axis=(0, 2), keepdims=True)
    )
    el = jnp.exp(lf - m)
    er = jnp.exp(rf - m)
    s = el.sum(axis=(0, 2), keepdims=True) + er.sum(axis=(0, 2), keepdims=True)
    el = el / s
    er = er / s

    @pl.when(my_x == 0)
    def _():
        for j in range(NB):
            o_ref[:, j * VB:(j + 1) * VB] = el[j]
            o_ref[:, VH + j * VB:VH + (j + 1) * VB] = er[j]

    @pl.when(my_x != 0)
    def _():
        for j in range(NB):
            o_ref[:, j * VB:(j + 1) * VB] = er[j]
            o_ref[:, VH + j * VB:VH + (j + 1) * VB] = el[j]


def _softmax(L, R):
    return pl.pallas_call(
        _softmax_body,
        grid=(T // R_BLK,),
        in_specs=[
            pl.BlockSpec((NB, R_BLK, VB), lambda i: (0, i, 0)),
            pl.BlockSpec((NB, R_BLK, VB), lambda i: (0, i, 0)),
        ],
        out_specs=pl.BlockSpec((R_BLK, 2 * VH), lambda i: (i, 0)),
        out_shape=jax.ShapeDtypeStruct((T, 2 * VH), jnp.float32),
    )(L, R)


def kernel(x, W):
    xb = _cast_x(x)
    L, R = _fused_gemm_exchange(xb, W)
    return _softmax(L, R)
